# baseline (speedup 1.0000x reference)
import functools

import jax
import jax.numpy as jnp
from jax import lax
from jax.experimental import pallas as pl
from jax.experimental.pallas import tpu as pltpu
from jax.experimental.pallas import tpu_sc as plsc

V = 10000
E = 64
BS = 16384
XMAX = 100.0
ALPHA = 0.75

NC = 2
NS = 16
L = 16
NW = NC * NS
BPW = BS // NW
HALF = BPW // 2
CH = 128
WV = 32
NWAVE = HALF // WV

_LN2 = 0.6931471805599453
_LNXMAX = 4.605170185988092


def _vlog(x):
    bits = plsc.bitcast(x, jnp.int32)
    e = ((bits >> 23) & 255) - 127
    m = plsc.bitcast((bits & 0x007FFFFF) | 0x3F800000, jnp.float32)
    t = (m - 1.0) / (m + 1.0)
    t2 = t * t
    lnm = 2.0 * t * (1.0 + t2 * (1.0 / 3 + t2 * (1.0 / 5 + t2 * (1.0 / 7))))
    return e.astype(jnp.float32) * _LN2 + lnm


def _sc_body(word_h, ctx_h, tab_h, bw_h, bc_h, comat_h, out_h,
             word_v, ctx_v, bw_v, bc_v, wrows, crows, co_v,
             strip_a, strip_b, idx_a, idx_b, out_v, sem, semr, semb):
    wid = lax.axis_index("s") * NC + lax.axis_index("c")
    base = wid * BPW

    pltpu.sync_copy(word_h.at[pl.ds(base, BPW)], word_v)
    pltpu.sync_copy(ctx_h.at[pl.ds(base, BPW)], ctx_v)
    bias_cps = [pltpu.async_copy(bw_h, bw_v, semb),
                pltpu.async_copy(bc_h, bc_v, semb)]

    lane = lax.iota(jnp.int32, L)
    zero = jnp.zeros((L,), jnp.int32)
    acc = jnp.zeros((L,), jnp.float32)

    for half in range(2):
        hb = half * HALF
        row_copies = []
        for j in range(HALF // CH):
            sl = pl.ds(hb + j * CH, CH)
            dsl = pl.ds(j * CH, CH)
            row_copies.append(
                pltpu.async_copy(tab_h.at[word_v.at[sl]], wrows.at[dsl], semr))
            row_copies.append(
                pltpu.async_copy(tab_h.at[ctx_v.at[sl]], crows.at[dsl], semr))

        if half == 0:
            for cp in bias_cps:
                cp.wait()
        for cp in row_copies:
            cp.wait()

        def group(g, acc_in):
            rid = g * L + lane
            dot = jnp.zeros((L,), jnp.float32)
            for e in range(E):
                ev = jnp.full((L,), e, jnp.int32)
                wv = plsc.load_gather(wrows, [rid, ev])
                cv = plsc.load_gather(crows, [rid, ev + E])
                dot = dot + wv * cv
            sl = pl.ds(hb + g * L, L)
            bwg = plsc.load_gather(bw_v, [word_v[sl]])
            bcg = plsc.load_gather(bc_v, [ctx_v[sl]])
            co = co_v[sl]
            lnco = _vlog(co)
            wgt = jnp.where(co < XMAX,
                            jnp.exp(ALPHA * (lnco - _LNXMAX)),
                            jnp.ones_like(co))
            d = dot + bwg + bcg - lnco
            return acc_in + d * d * wgt

        def wave_pair(p, acc_p):
            def fire(buf, ibuf, w):
                wbase = hb + w * WV
                css = []
                cps = []
                for g in range(WV // L):
                    ws = word_v[pl.ds(wbase + g * L, L)]
                    plsc.store_scatter(ibuf, [(lane + g * L) * 8], ws)
                    cs = ctx_v[pl.ds(wbase + g * L, L)]
                    css.append(cs)
                    for j in range(L):
                        cb = pl.multiple_of((cs[j] >> 7) << 7, 128)
                        stripe = comat_h.at[:, pl.ds(cb, 128)]
                        idx = ibuf.at[pl.ds((g * L + j) * 8, 1)]
                        cps.append(pltpu.async_copy(
                            stripe.at[idx], buf.at[g * L + j], sem))
                return css, cps

            def extract(buf, css, cps, w):
                for cp in cps:
                    cp.wait()
                for g in range(WV // L):
                    gv = jnp.full((L,), g * L, jnp.int32) + lane
                    co_v[pl.ds(hb + w * WV + g * L, L)] = plsc.load_gather(
                        buf, [gv, zero, css[g] & 127])

            wa = p * 2
            csa, cpsa = fire(strip_a, idx_a, wa)
            csb, cpsb = fire(strip_b, idx_b, wa + 1)
            extract(strip_a, csa, cpsa, wa)
            acc_p = group(4 * p, acc_p)
            acc_p = group(4 * p + 1, acc_p)
            extract(strip_b, csb, cpsb, wa + 1)
            acc_p = group(4 * p + 2, acc_p)
            return group(4 * p + 3, acc_p)

        acc = lax.fori_loop(0, NWAVE // 2, wave_pair, acc)

    out_v[pl.ds(0, L)] = acc
    pltpu.sync_copy(out_v, out_h.at[pl.ds(wid * L, L)])


_sc_loss = functools.partial(
    pl.kernel,
    out_type=jax.ShapeDtypeStruct((NW * L,), jnp.float32),
    mesh=plsc.VectorSubcoreMesh(core_axis_name="c", subcore_axis_name="s",
                                num_cores=NC, num_subcores=NS),
    compiler_params=pltpu.CompilerParams(needs_layout_passes=False),
    scratch_types=[
        pltpu.VMEM((BPW,), jnp.int32),
        pltpu.VMEM((BPW,), jnp.int32),
        pltpu.VMEM((V,), jnp.float32),
        pltpu.VMEM((V,), jnp.float32),
        pltpu.VMEM((HALF, 2 * E), jnp.float32),
        pltpu.VMEM((HALF, 2 * E), jnp.float32),
        pltpu.VMEM((BPW,), jnp.float32),
        pltpu.VMEM((WV, 1, 128), jnp.float32),
        pltpu.VMEM((WV, 1, 128), jnp.float32),
        pltpu.VMEM((WV * 8,), jnp.int32),
        pltpu.VMEM((WV * 8,), jnp.int32),
        pltpu.VMEM((L,), jnp.float32),
        pltpu.SemaphoreType.DMA,
        pltpu.SemaphoreType.DMA,
        pltpu.SemaphoreType.DMA,
    ],
)(_sc_body)


def kernel(word, context, Wword, Wctx, bword, bctx, comat):
    word = word.astype(jnp.int32)
    context = context.astype(jnp.int32)
    table = jnp.concatenate([Wword, Wctx], axis=1)
    parts = _sc_loss(word, context, table,
                     bword.reshape(-1), bctx.reshape(-1), comat)
    return jnp.sum(parts)


# speedup vs baseline: 1.1208x; 1.1208x over previous
import functools

import jax
import jax.numpy as jnp
from jax import lax
from jax.experimental import pallas as pl
from jax.experimental.pallas import tpu as pltpu
from jax.experimental.pallas import tpu_sc as plsc

V = 10000
E = 64
BS = 16384
XMAX = 100.0
ALPHA = 0.75

NC = 2
NS = 16
L = 16
NW = NC * NS
BPW = BS // NW
HALF = BPW // 2
CH = 128
WV = 64
NWAVE = HALF // WV

_LN2 = 0.6931471805599453
_LNXMAX = 4.605170185988092


def _vlog(x):
    bits = plsc.bitcast(x, jnp.int32)
    e = ((bits >> 23) & 255) - 127
    m = plsc.bitcast((bits & 0x007FFFFF) | 0x3F800000, jnp.float32)
    t = (m - 1.0) / (m + 1.0)
    t2 = t * t
    lnm = 2.0 * t * (1.0 + t2 * (1.0 / 3 + t2 * (1.0 / 5 + t2 * (1.0 / 7))))
    return e.astype(jnp.float32) * _LN2 + lnm


def _sc_body(word_h, ctx_h, tab_h, bw_h, bc_h, comat_h, out_h,
             word_v, ctx_v, bw_v, bc_v, wrows, crows, co_v,
             strip_a, strip_b, idx_a, idx_b, out_v, sem, semr, semb):
    wid = lax.axis_index("s") * NC + lax.axis_index("c")
    base = wid * BPW

    pltpu.sync_copy(word_h.at[pl.ds(base, BPW)], word_v)
    pltpu.sync_copy(ctx_h.at[pl.ds(base, BPW)], ctx_v)
    bias_cps = [pltpu.async_copy(bw_h, bw_v, semb),
                pltpu.async_copy(bc_h, bc_v, semb)]

    lane = lax.iota(jnp.int32, L)
    zero = jnp.zeros((L,), jnp.int32)
    acc = jnp.zeros((L,), jnp.float32)

    for half in range(2):
        hb = half * HALF
        row_copies = []
        for j in range(HALF // CH):
            sl = pl.ds(hb + j * CH, CH)
            dsl = pl.ds(j * CH, CH)
            row_copies.append(
                pltpu.async_copy(tab_h.at[word_v.at[sl]], wrows.at[dsl], semr))
            row_copies.append(
                pltpu.async_copy(tab_h.at[ctx_v.at[sl]], crows.at[dsl], semr))

        def group(g, acc_in):
            rid = g * L + lane
            dot = jnp.zeros((L,), jnp.float32)
            for e in range(E):
                ev = jnp.full((L,), e, jnp.int32)
                wv = plsc.load_gather(wrows, [rid, ev])
                cv = plsc.load_gather(crows, [rid, ev + E])
                dot = dot + wv * cv
            sl = pl.ds(hb + g * L, L)
            bwg = plsc.load_gather(bw_v, [word_v[sl]])
            bcg = plsc.load_gather(bc_v, [ctx_v[sl]])
            co = co_v[sl]
            lnco = _vlog(co)
            wgt = jnp.where(co < XMAX,
                            jnp.exp(ALPHA * (lnco - _LNXMAX)),
                            jnp.ones_like(co))
            d = dot + bwg + bcg - lnco
            return acc_in + d * d * wgt

        def wave_pair(p, acc_p):
            def fire(buf, ibuf, w):
                wbase = hb + w * WV
                css = []
                cps = []
                for g in range(WV // L):
                    ws = word_v[pl.ds(wbase + g * L, L)]
                    plsc.store_scatter(ibuf, [(lane + g * L) * 8], ws)
                    cs = ctx_v[pl.ds(wbase + g * L, L)]
                    css.append(cs)
                    for j in range(L):
                        cb = pl.multiple_of((cs[j] >> 7) << 7, 128)
                        stripe = comat_h.at[:, pl.ds(cb, 128)]
                        idx = ibuf.at[pl.ds((g * L + j) * 8, 1)]
                        cps.append(pltpu.async_copy(
                            stripe.at[idx], buf.at[g * L + j], sem))
                return css, cps

            def extract(buf, css, cps, w):
                for cp in cps:
                    cp.wait()
                for g in range(WV // L):
                    gv = jnp.full((L,), g * L, jnp.int32) + lane
                    co_v[pl.ds(hb + w * WV + g * L, L)] = plsc.load_gather(
                        buf, [gv, zero, css[g] & 127])

            wa = p * 2
            csa, cpsa = fire(strip_a, idx_a, wa)
            csb, cpsb = fire(strip_b, idx_b, wa + 1)
            extract(strip_a, csa, cpsa, wa)
            extract(strip_b, csb, cpsb, wa + 1)
            return acc_p

        acc = lax.fori_loop(0, NWAVE // 2, wave_pair, acc)

        if half == 0:
            for cp in bias_cps:
                cp.wait()
        for cp in row_copies:
            cp.wait()

        acc = lax.fori_loop(0, HALF // L, group, acc)

    out_v[pl.ds(0, L)] = acc
    pltpu.sync_copy(out_v, out_h.at[pl.ds(wid * L, L)])


_sc_loss = functools.partial(
    pl.kernel,
    out_type=jax.ShapeDtypeStruct((NW * L,), jnp.float32),
    mesh=plsc.VectorSubcoreMesh(core_axis_name="c", subcore_axis_name="s",
                                num_cores=NC, num_subcores=NS),
    compiler_params=pltpu.CompilerParams(needs_layout_passes=False),
    scratch_types=[
        pltpu.VMEM((BPW,), jnp.int32),
        pltpu.VMEM((BPW,), jnp.int32),
        pltpu.VMEM((V,), jnp.float32),
        pltpu.VMEM((V,), jnp.float32),
        pltpu.VMEM((HALF, 2 * E), jnp.float32),
        pltpu.VMEM((HALF, 2 * E), jnp.float32),
        pltpu.VMEM((BPW,), jnp.float32),
        pltpu.VMEM((WV, 1, 128), jnp.float32),
        pltpu.VMEM((WV, 1, 128), jnp.float32),
        pltpu.VMEM((WV * 8,), jnp.int32),
        pltpu.VMEM((WV * 8,), jnp.int32),
        pltpu.VMEM((L,), jnp.float32),
        pltpu.SemaphoreType.DMA,
        pltpu.SemaphoreType.DMA,
        pltpu.SemaphoreType.DMA,
    ],
)(_sc_body)


def kernel(word, context, Wword, Wctx, bword, bctx, comat):
    word = word.astype(jnp.int32)
    context = context.astype(jnp.int32)
    table = jnp.concatenate([Wword, Wctx], axis=1)
    parts = _sc_loss(word, context, table,
                     bword.reshape(-1), bctx.reshape(-1), comat)
    return jnp.sum(parts)


# speedup vs baseline: 1.1650x; 1.0394x over previous
import functools

import jax
import jax.numpy as jnp
from jax import lax
from jax.experimental import pallas as pl
from jax.experimental.pallas import tpu as pltpu
from jax.experimental.pallas import tpu_sc as plsc

V = 10000
E = 64
BS = 16384
XMAX = 100.0
ALPHA = 0.75

NC = 2
NS = 16
L = 16
NW = NC * NS
BPW = BS // NW
HALF = BPW // 2
CH = 128
WV = 64
NWAVE = HALF // WV

_LN2 = 0.6931471805599453
_LNXMAX = 4.605170185988092


def _vlog(x):
    bits = plsc.bitcast(x, jnp.int32)
    e = ((bits >> 23) & 255) - 127
    m = plsc.bitcast((bits & 0x007FFFFF) | 0x3F800000, jnp.float32)
    t = (m - 1.0) / (m + 1.0)
    t2 = t * t
    lnm = 2.0 * t * (1.0 + t2 * (1.0 / 3 + t2 * (1.0 / 5 + t2 * (1.0 / 7))))
    return e.astype(jnp.float32) * _LN2 + lnm


def _sc_co_body(word_h, ctx_h, comat_h, co_out_h,
                word_v, ctx_v, co_v, strip_a, strip_b, idx_a, idx_b, sem):
    wid = lax.axis_index("s") * NC + lax.axis_index("c")
    base = wid * BPW

    pltpu.sync_copy(word_h.at[pl.ds(base, BPW)], word_v)
    pltpu.sync_copy(ctx_h.at[pl.ds(base, BPW)], ctx_v)

    lane = lax.iota(jnp.int32, L)
    zero = jnp.zeros((L,), jnp.int32)

    def fire(buf, ibuf, w):
        wbase = w * WV
        css = []
        cps = []
        for g in range(WV // L):
            ws = word_v[pl.ds(wbase + g * L, L)]
            plsc.store_scatter(ibuf, [(lane + g * L) * 8], ws)
            cs = ctx_v[pl.ds(wbase + g * L, L)]
            css.append(cs)
            for j in range(L):
                cb = pl.multiple_of((cs[j] >> 7) << 7, 128)
                stripe = comat_h.at[:, pl.ds(cb, 128)]
                idx = ibuf.at[pl.ds((g * L + j) * 8, 1)]
                cps.append(pltpu.async_copy(
                    stripe.at[idx], buf.at[g * L + j], sem))
        return css, cps

    def extract(buf, css, w):
        for g in range(WV // L):
            gv = jnp.full((L,), g * L, jnp.int32) + lane
            co_v[pl.ds(w * WV + g * L, L)] = plsc.load_gather(
                buf, [gv, zero, css[g] & 127])

    def wave_pair(p, carry):
        wa = p * 2
        csa, cpsa = fire(strip_a, idx_a, wa)
        csb, cpsb = fire(strip_b, idx_b, wa + 1)
        for cp in cpsa:
            cp.wait()
        extract(strip_a, csa, wa)
        for cp in cpsb:
            cp.wait()
        extract(strip_b, csb, wa + 1)
        return carry

    lax.fori_loop(0, BPW // WV // 2, wave_pair, 0)
    pltpu.sync_copy(co_v, co_out_h.at[pl.ds(base, BPW)])


_sc_co = functools.partial(
    pl.kernel,
    out_type=jax.ShapeDtypeStruct((BS,), jnp.float32),
    mesh=plsc.VectorSubcoreMesh(core_axis_name="c", subcore_axis_name="s",
                                num_cores=NC, num_subcores=NS),
    compiler_params=pltpu.CompilerParams(needs_layout_passes=False),
    scratch_types=[
        pltpu.VMEM((BPW,), jnp.int32),
        pltpu.VMEM((BPW,), jnp.int32),
        pltpu.VMEM((BPW,), jnp.float32),
        pltpu.VMEM((WV, 1, 128), jnp.float32),
        pltpu.VMEM((WV, 1, 128), jnp.float32),
        pltpu.VMEM((WV * 8,), jnp.int32),
        pltpu.VMEM((WV * 8,), jnp.int32),
        pltpu.SemaphoreType.DMA,
    ],
)(_sc_co_body)


def _sc_body(word_h, ctx_h, tab_h, bw_h, bc_h, co_h, out_h,
             word_v, ctx_v, bw_v, bc_v, rows_a, rows_b, co_v,
             out_v, sem, semr, semb):
    wid = lax.axis_index("s") * NC + lax.axis_index("c")
    base = wid * BPW

    pltpu.sync_copy(word_h.at[pl.ds(base, BPW)], word_v)
    pltpu.sync_copy(ctx_h.at[pl.ds(base, BPW)], ctx_v)
    co_cp = pltpu.async_copy(co_h.at[pl.ds(base, BPW)], co_v, sem)
    bias_cps = [pltpu.async_copy(bw_h, bw_v, semb),
                pltpu.async_copy(bc_h, bc_v, semb)]

    lane = lax.iota(jnp.int32, L)
    acc = jnp.zeros((L,), jnp.float32)

    NQ = BPW // CH

    def fire_rows(q, buf):
        sl = pl.ds(q * CH, CH)
        return [pltpu.async_copy(tab_h.at[word_v.at[sl]], buf.at[0], semr),
                pltpu.async_copy(tab_h.at[ctx_v.at[sl]], buf.at[1], semr)]

    pend = {0: fire_rows(0, rows_a), 1: fire_rows(1, rows_b)}

    co_cp.wait()
    for cp in bias_cps:
        cp.wait()

    for q in range(NQ):
        buf = rows_a if q % 2 == 0 else rows_b
        for cp in pend[q]:
            cp.wait()

        def group(g, acc_in, qb=q * CH, buf=buf):
            rid = g * L + lane
            sz = jnp.zeros((L,), jnp.int32)
            so = jnp.full((L,), 1, jnp.int32)
            dot = jnp.zeros((L,), jnp.float32)
            for e in range(E):
                ev = jnp.full((L,), e, jnp.int32)
                wv = plsc.load_gather(buf, [sz, rid, ev])
                cv = plsc.load_gather(buf, [so, rid, ev + E])
                dot = dot + wv * cv
            sl = pl.ds(qb + g * L, L)
            bwg = plsc.load_gather(bw_v, [word_v[sl]])
            bcg = plsc.load_gather(bc_v, [ctx_v[sl]])
            co = co_v[sl]
            lnco = _vlog(co)
            wgt = jnp.where(co < XMAX,
                            jnp.exp(ALPHA * (lnco - _LNXMAX)),
                            jnp.ones_like(co))
            d = dot + bwg + bcg - lnco
            return acc_in + d * d * wgt

        if q + 2 < NQ:
            pend[q + 2] = fire_rows(q + 2, buf)

        acc = lax.fori_loop(0, CH // L, lambda g, a, fn=group: fn(g, a), acc)

    out_v[pl.ds(0, L)] = acc
    pltpu.sync_copy(out_v, out_h.at[pl.ds(wid * L, L)])


_sc_loss = functools.partial(
    pl.kernel,
    out_type=jax.ShapeDtypeStruct((NW * L,), jnp.float32),
    mesh=plsc.VectorSubcoreMesh(core_axis_name="c", subcore_axis_name="s",
                                num_cores=NC, num_subcores=NS),
    compiler_params=pltpu.CompilerParams(needs_layout_passes=False),
    scratch_types=[
        pltpu.VMEM((BPW,), jnp.int32),
        pltpu.VMEM((BPW,), jnp.int32),
        pltpu.VMEM((V,), jnp.float32),
        pltpu.VMEM((V,), jnp.float32),
        pltpu.VMEM((2, CH, 2 * E), jnp.float32),
        pltpu.VMEM((2, CH, 2 * E), jnp.float32),
        pltpu.VMEM((BPW,), jnp.float32),
        pltpu.VMEM((L,), jnp.float32),
        pltpu.SemaphoreType.DMA,
        pltpu.SemaphoreType.DMA,
        pltpu.SemaphoreType.DMA,
    ],
)(_sc_body)


def kernel(word, context, Wword, Wctx, bword, bctx, comat):
    word = word.astype(jnp.int32)
    context = context.astype(jnp.int32)
    co = _sc_co(word, context, comat)
    table = jnp.concatenate([Wword, Wctx], axis=1)
    parts = _sc_loss(word, context, table,
                     bword.reshape(-1), bctx.reshape(-1), co)
    return jnp.sum(parts)


# speedup vs baseline: 1.1735x; 1.0073x over previous
import functools

import jax
import jax.numpy as jnp
from jax import lax
from jax.experimental import pallas as pl
from jax.experimental.pallas import tpu as pltpu
from jax.experimental.pallas import tpu_sc as plsc

V = 10000
E = 64
BS = 16384
XMAX = 100.0
ALPHA = 0.75

NC = 2
NS = 16
L = 16
NW = NC * NS
BPW = BS // NW
HALF = BPW // 2
CH = 128
WV = 64
NWAVE = HALF // WV

_LN2 = 0.6931471805599453
_LNXMAX = 4.605170185988092


def _vlog(x):
    bits = plsc.bitcast(x, jnp.int32)
    e = ((bits >> 23) & 255) - 127
    m = plsc.bitcast((bits & 0x007FFFFF) | 0x3F800000, jnp.float32)
    t = (m - 1.0) / (m + 1.0)
    t2 = t * t
    lnm = 2.0 * t * (1.0 + t2 * (1.0 / 3 + t2 * (1.0 / 5 + t2 * (1.0 / 7))))
    return e.astype(jnp.float32) * _LN2 + lnm


def _sc_co_body(word_h, ctx_h, comat_h, co_out_h,
                word_v, ctx_v, co_v, strip_a, strip_b, idx_a, idx_b, sem):
    wid = lax.axis_index("s") * NC + lax.axis_index("c")
    base = wid * BPW

    pltpu.sync_copy(word_h.at[pl.ds(base, BPW)], word_v)
    pltpu.sync_copy(ctx_h.at[pl.ds(base, BPW)], ctx_v)

    lane = lax.iota(jnp.int32, L)
    zero = jnp.zeros((L,), jnp.int32)

    def fire(buf, ibuf, w):
        wbase = w * WV
        css = []
        cps = []
        for g in range(WV // L):
            ws = word_v[pl.ds(wbase + g * L, L)]
            plsc.store_scatter(ibuf, [(lane + g * L) * 8], ws)
            cs = ctx_v[pl.ds(wbase + g * L, L)]
            css.append(cs)
            for j in range(L):
                cb = pl.multiple_of((cs[j] >> 7) << 7, 128)
                stripe = comat_h.at[:, pl.ds(cb, 128)]
                idx = ibuf.at[pl.ds((g * L + j) * 8, 1)]
                cps.append(pltpu.async_copy(
                    stripe.at[idx], buf.at[g * L + j], sem))
        return css, cps

    def extract(buf, css, w):
        for g in range(WV // L):
            gv = jnp.full((L,), g * L, jnp.int32) + lane
            co_v[pl.ds(w * WV + g * L, L)] = plsc.load_gather(
                buf, [gv, zero, css[g] & 127])

    def wave_pair(p, carry):
        wa = p * 2
        csa, cpsa = fire(strip_a, idx_a, wa)
        csb, cpsb = fire(strip_b, idx_b, wa + 1)
        for cp in cpsa:
            cp.wait()
        extract(strip_a, csa, wa)
        for cp in cpsb:
            cp.wait()
        extract(strip_b, csb, wa + 1)
        return carry

    lax.fori_loop(0, BPW // WV // 2, wave_pair, 0)
    pltpu.sync_copy(co_v, co_out_h.at[pl.ds(base, BPW)])


_sc_co = functools.partial(
    pl.kernel,
    out_type=jax.ShapeDtypeStruct((BS,), jnp.float32),
    mesh=plsc.VectorSubcoreMesh(core_axis_name="c", subcore_axis_name="s",
                                num_cores=NC, num_subcores=NS),
    compiler_params=pltpu.CompilerParams(needs_layout_passes=False),
    scratch_types=[
        pltpu.VMEM((BPW,), jnp.int32),
        pltpu.VMEM((BPW,), jnp.int32),
        pltpu.VMEM((BPW,), jnp.float32),
        pltpu.VMEM((WV, 1, 128), jnp.float32),
        pltpu.VMEM((WV, 1, 128), jnp.float32),
        pltpu.VMEM((WV * 8,), jnp.int32),
        pltpu.VMEM((WV * 8,), jnp.int32),
        pltpu.SemaphoreType.DMA,
    ],
)(_sc_co_body)


def _sc_body(word_h, ctx_h, tab_h, bw_h, bc_h, co_h, out_h,
             word_v, ctx_v, bw_v, bc_v, rows_a, rows_b, co_v,
             out_v, sem, semr, semb):
    wid = lax.axis_index("s") * NC + lax.axis_index("c")
    base = wid * BPW

    pltpu.sync_copy(word_h.at[pl.ds(base, BPW)], word_v)
    pltpu.sync_copy(ctx_h.at[pl.ds(base, BPW)], ctx_v)
    co_cp = pltpu.async_copy(co_h.at[pl.ds(base, BPW)], co_v, sem)
    bias_cps = [pltpu.async_copy(bw_h, bw_v, semb),
                pltpu.async_copy(bc_h, bc_v, semb)]

    lane = lax.iota(jnp.int32, L)
    acc = jnp.zeros((L,), jnp.float32)

    NQ = BPW // CH

    def fire_rows(q, buf):
        sl = pl.ds(q * CH, CH)
        return [pltpu.async_copy(tab_h.at[word_v.at[sl]], buf.at[0], semr),
                pltpu.async_copy(tab_h.at[ctx_v.at[sl]], buf.at[1], semr)]

    pend = {0: fire_rows(0, rows_a), 1: fire_rows(1, rows_b)}

    co_cp.wait()
    for cp in bias_cps:
        cp.wait()

    for q in range(NQ):
        buf = rows_a if q % 2 == 0 else rows_b
        for cp in pend[q]:
            cp.wait()

        def group(g, acc_in, qb=q * CH, buf=buf):
            rid = g * L + lane
            sz = jnp.zeros((L,), jnp.int32)
            so = jnp.full((L,), 1, jnp.int32)
            dot = jnp.zeros((L,), jnp.float32)
            for e in range(E):
                ev = jnp.full((L,), e, jnp.int32)
                wv = plsc.load_gather(buf, [sz, rid, ev])
                cv = plsc.load_gather(buf, [so, rid, ev + E])
                dot = dot + wv * cv
            sl = pl.ds(qb + g * L, L)
            bwg = plsc.load_gather(bw_v, [word_v[sl]])
            bcg = plsc.load_gather(bc_v, [ctx_v[sl]])
            co = co_v[sl]
            lnco = _vlog(co)
            wgt = jnp.where(co < XMAX,
                            jnp.exp(ALPHA * (lnco - _LNXMAX)),
                            jnp.ones_like(co))
            d = dot + bwg + bcg - lnco
            return acc_in + d * d * wgt

        acc = lax.fori_loop(0, CH // L, lambda g, a, fn=group: fn(g, a), acc)

        if q + 2 < NQ:
            pend[q + 2] = fire_rows(q + 2, buf)

    out_v[pl.ds(0, L)] = acc
    pltpu.sync_copy(out_v, out_h.at[pl.ds(wid * L, L)])


_sc_loss = functools.partial(
    pl.kernel,
    out_type=jax.ShapeDtypeStruct((NW * L,), jnp.float32),
    mesh=plsc.VectorSubcoreMesh(core_axis_name="c", subcore_axis_name="s",
                                num_cores=NC, num_subcores=NS),
    compiler_params=pltpu.CompilerParams(needs_layout_passes=False),
    scratch_types=[
        pltpu.VMEM((BPW,), jnp.int32),
        pltpu.VMEM((BPW,), jnp.int32),
        pltpu.VMEM((V,), jnp.float32),
        pltpu.VMEM((V,), jnp.float32),
        pltpu.VMEM((2, CH, 2 * E), jnp.float32),
        pltpu.VMEM((2, CH, 2 * E), jnp.float32),
        pltpu.VMEM((BPW,), jnp.float32),
        pltpu.VMEM((L,), jnp.float32),
        pltpu.SemaphoreType.DMA,
        pltpu.SemaphoreType.DMA,
        pltpu.SemaphoreType.DMA,
    ],
)(_sc_body)


def kernel(word, context, Wword, Wctx, bword, bctx, comat):
    word = word.astype(jnp.int32)
    context = context.astype(jnp.int32)
    co = _sc_co(word, context, comat)
    table = jnp.concatenate([Wword, Wctx], axis=1)
    parts = _sc_loss(word, context, table,
                     bword.reshape(-1), bctx.reshape(-1), co)
    return jnp.sum(parts)
